# Initial kernel scaffold; baseline (speedup 1.0000x reference)
import jax
import jax.numpy as jnp
from jax.experimental import pallas as pl


def kernel(pid_fp_idx, pid_ipc_idx, patent_company_idx, company_industry_idx, company_patent_idx, company_table, patent_table, fp_table, ipc_table, industry_table):
    raise NotImplementedError("write your pallas kernel here")


# speedup vs baseline: 4.6080x; 4.6080x over previous
import functools

import jax
import jax.numpy as jnp
from jax import lax
from jax.experimental import pallas as pl
from jax.experimental.pallas import tpu as pltpu
from jax.experimental.pallas import tpu_sc as plsc

P = 100000
C = 20000
D = 128
B = 160
NPB = P // B
NCB = C // B
NW = 32
NV = D // 16


def _sc_kernel(fp_t, ipc_t, pc_t, ci_t, cp_t,
               company_table, patent_table, fp_table, ipc_table,
               industry_table, out,
               idx_v, rows_v, acc_v, base_v, sem):
    nc = 2
    wid = lax.axis_index("s") * nc + lax.axis_index("c")

    def accumulate_rows(dst, src):
        def row(i, _):
            for v in range(NV):
                sl = pl.ds(v * 16, 16)
                dst[i, sl] = dst[i, sl] + src[i, sl]
            return 0
        lax.fori_loop(0, B, row, 0)

    def finalize(scale):
        def row(i, _):
            for v in range(NV):
                sl = pl.ds(v * 16, 16)
                acc_v[i, sl] = base_v[i, sl] + acc_v[i, sl] * scale
            return 0
        lax.fori_loop(0, B, row, 0)

    def gather_slot(idx_arr, off, table, dst):
        pltpu.sync_copy(idx_arr.at[pl.ds(off, B)], idx_v)
        pltpu.async_copy(table.at[idx_v], dst, sem).wait()

    def pblock(t, _):
        b = wid + t * NW

        @pl.when(b < NPB)
        def _():
            base = b * B
            pltpu.sync_copy(patent_table.at[pl.ds(base, B)], base_v)
            gather_slot(fp_t, base, fp_table, acc_v)
            for j in range(1, 4):
                gather_slot(fp_t, j * P + base, fp_table, rows_v)
                accumulate_rows(acc_v, rows_v)
            for j in range(2):
                gather_slot(ipc_t, j * P + base, ipc_table, rows_v)
                accumulate_rows(acc_v, rows_v)
            for j in range(2):
                gather_slot(pc_t, j * P + base, company_table, rows_v)
                accumulate_rows(acc_v, rows_v)
            finalize(0.125)
            pltpu.sync_copy(acc_v, out.at[pl.ds(base, B)])
        return 0

    def cblock(t, _):
        b = wid + t * NW

        @pl.when(b < NCB)
        def _():
            base = b * B
            pltpu.sync_copy(company_table.at[pl.ds(base, B)], base_v)
            gather_slot(ci_t, base, industry_table, acc_v)
            gather_slot(ci_t, C + base, industry_table, rows_v)
            accumulate_rows(acc_v, rows_v)
            for j in range(16):
                gather_slot(cp_t, j * C + base, patent_table, rows_v)
                accumulate_rows(acc_v, rows_v)
            finalize(1.0 / 18.0)
            pltpu.sync_copy(acc_v, out.at[pl.ds(P + base, B)])
        return 0

    lax.fori_loop(0, pl.cdiv(NPB, NW), pblock, 0)
    lax.fori_loop(0, pl.cdiv(NCB, NW), cblock, 0)


def kernel(pid_fp_idx, pid_ipc_idx, patent_company_idx, company_industry_idx,
           company_patent_idx, company_table, patent_table, fp_table,
           ipc_table, industry_table):
    fp_t = pid_fp_idx.T.reshape(-1).astype(jnp.int32)
    ipc_t = pid_ipc_idx.T.reshape(-1).astype(jnp.int32)
    pc_t = patent_company_idx.T.reshape(-1).astype(jnp.int32)
    ci_t = company_industry_idx.T.reshape(-1).astype(jnp.int32)
    cp_t = company_patent_idx.T.reshape(-1).astype(jnp.int32)

    mesh = plsc.VectorSubcoreMesh(core_axis_name="c", subcore_axis_name="s")
    run = pl.kernel(
        _sc_kernel,
        out_type=jax.ShapeDtypeStruct((P + C, D), jnp.float32),
        mesh=mesh,
        scratch_types=[
            pltpu.VMEM((B,), jnp.int32),
            pltpu.VMEM((B, D), jnp.float32),
            pltpu.VMEM((B, D), jnp.float32),
            pltpu.VMEM((B, D), jnp.float32),
            pltpu.SemaphoreType.DMA,
        ],
    )
    return run(fp_t, ipc_t, pc_t, ci_t, cp_t, company_table, patent_table,
               fp_table, ipc_table, industry_table)


# speedup vs baseline: 9.0213x; 1.9578x over previous
import jax
import jax.numpy as jnp
from jax import lax
from jax.experimental import pallas as pl
from jax.experimental.pallas import tpu as pltpu
from jax.experimental.pallas import tpu_sc as plsc

P = 100000
C = 20000
D = 128
B = 160
NPB = P // B
NCB = C // B
NW = 32
NV = D // 16


def _sc_kernel(fp_t, ipc_t, pc_t, ci_t, cp_t,
               company_table, patent_table, fp_table, ipc_table,
               industry_table, out,
               i_first, i0, i1, r0, r1, acc_v, base_v,
               isem_f, isem0, isem1, gsem0, gsem1, sema, bsem, outsem):
    nc = 2
    wid = lax.axis_index("s") * nc + lax.axis_index("c")
    i_cyc = (i0, i1)
    isems = (isem0, isem1)
    rbuf = (r0, r1)
    gsems = (gsem0, gsem1)

    def accumulate_rows(src):
        def row(i2, _):
            for u in range(2):
                i = i2 * 2 + u
                for v in range(NV):
                    sl = pl.ds(v * 16, 16)
                    plsc.addupdate(acc_v.at[i, sl], src[i, sl])
            return 0
        lax.fori_loop(0, B // 2, row, 0)

    def final_rows(src, scale):
        def row(i, _):
            for v in range(NV):
                sl = pl.ds(v * 16, 16)
                acc_v[i, sl] = base_v[i, sl] + (acc_v[i, sl] + src[i, sl]) * scale
            return 0
        lax.fori_loop(0, B, row, 0)

    def phase(nb, nt, slots, base_tab, out_off, scale, first_phase):
        n = len(slots)

        arr0, _, off0 = slots[0]

        @pl.when(wid < nb)
        def _():
            pltpu.async_copy(arr0.at[pl.ds(off0 + wid * B, B)], i_first, isem_f)

        def block(t, _):
            b = wid + t * NW

            @pl.when(b < nb)
            def _():
                base = b * B
                pltpu.async_copy(base_tab.at[pl.ds(base, B)], base_v, bsem)
                if first_phase:
                    @pl.when(t > 0)
                    def _():
                        pltpu.make_async_copy(acc_v, out.at[pl.ds(0, B)], outsem).wait()
                else:
                    pltpu.make_async_copy(acc_v, out.at[pl.ds(0, B)], outsem).wait()
                pltpu.make_async_copy(arr0.at[pl.ds(off0 + base, B)], i_first, isem_f).wait()
                _, tab0, _ = slots[0]
                pltpu.async_copy(tab0.at[i_first], acc_v, sema)
                arr1, _, offs1 = slots[1]
                pltpu.async_copy(arr1.at[pl.ds(offs1 + base, B)], i_cyc[1], isems[1])

                for k in range(1, n):
                    kb = k % 2
                    arrk, tabk, offk = slots[k]
                    pltpu.make_async_copy(
                        arrk.at[pl.ds(offk + base, B)], i_cyc[kb], isems[kb]).wait()
                    pltpu.async_copy(tabk.at[i_cyc[kb]], rbuf[kb], gsems[kb])
                    if k == 1:
                        pltpu.async_copy(
                            slots[2][0].at[pl.ds(slots[2][2] + base, B)],
                            i_cyc[0], isems[0])
                        pltpu.make_async_copy(tab0.at[i_first], acc_v, sema).wait()
                    else:
                        pkb = (k - 1) % 2
                        pltpu.make_async_copy(
                            slots[k - 1][1].at[i_cyc[pkb]], rbuf[pkb],
                            gsems[pkb]).wait()
                        if k + 1 < n:
                            arrn, _, offn = slots[k + 1]
                            pltpu.async_copy(
                                arrn.at[pl.ds(offn + base, B)],
                                i_cyc[(k + 1) % 2], isems[(k + 1) % 2])
                        else:
                            @pl.when(b + NW < nb)
                            def _():
                                pltpu.async_copy(
                                    arr0.at[pl.ds(off0 + (base + NW * B), B)],
                                    i_first, isem_f)
                        accumulate_rows(rbuf[pkb])

                lkb = (n - 1) % 2
                pltpu.make_async_copy(
                    slots[n - 1][1].at[i_cyc[lkb]], rbuf[lkb], gsems[lkb]).wait()
                pltpu.make_async_copy(base_tab.at[pl.ds(base, B)], base_v, bsem).wait()
                final_rows(rbuf[lkb], scale)
                pltpu.async_copy(acc_v, out.at[pl.ds(out_off + base, B)], outsem)
            return 0

        lax.fori_loop(0, nt, block, 0)

    p_slots = ([(fp_t, fp_table, j * P) for j in range(4)]
               + [(ipc_t, ipc_table, j * P) for j in range(2)]
               + [(pc_t, company_table, j * P) for j in range(2)])
    c_slots = ([(ci_t, industry_table, j * C) for j in range(2)]
               + [(cp_t, patent_table, j * C) for j in range(16)])

    phase(NPB, pl.cdiv(NPB, NW), p_slots, patent_table, 0, 0.125, True)
    phase(NCB, pl.cdiv(NCB, NW), c_slots, company_table, P, 1.0 / 18.0, False)
    pltpu.make_async_copy(acc_v, out.at[pl.ds(0, B)], outsem).wait()


def kernel(pid_fp_idx, pid_ipc_idx, patent_company_idx, company_industry_idx,
           company_patent_idx, company_table, patent_table, fp_table,
           ipc_table, industry_table):
    fp_t = pid_fp_idx.T.reshape(-1).astype(jnp.int32)
    ipc_t = pid_ipc_idx.T.reshape(-1).astype(jnp.int32)
    pc_t = patent_company_idx.T.reshape(-1).astype(jnp.int32)
    ci_t = company_industry_idx.T.reshape(-1).astype(jnp.int32)
    cp_t = company_patent_idx.T.reshape(-1).astype(jnp.int32)

    mesh = plsc.VectorSubcoreMesh(core_axis_name="c", subcore_axis_name="s")
    run = pl.kernel(
        _sc_kernel,
        out_type=jax.ShapeDtypeStruct((P + C, D), jnp.float32),
        mesh=mesh,
        scratch_types=[
            pltpu.VMEM((B,), jnp.int32),
            pltpu.VMEM((B,), jnp.int32),
            pltpu.VMEM((B,), jnp.int32),
            pltpu.VMEM((B, D), jnp.float32),
            pltpu.VMEM((B, D), jnp.float32),
            pltpu.VMEM((B, D), jnp.float32),
            pltpu.VMEM((B, D), jnp.float32),
            pltpu.SemaphoreType.DMA,
            pltpu.SemaphoreType.DMA,
            pltpu.SemaphoreType.DMA,
            pltpu.SemaphoreType.DMA,
            pltpu.SemaphoreType.DMA,
            pltpu.SemaphoreType.DMA,
            pltpu.SemaphoreType.DMA,
            pltpu.SemaphoreType.DMA,
        ],
    )
    return run(fp_t, ipc_t, pc_t, ci_t, cp_t, company_table, patent_table,
               fp_table, ipc_table, industry_table)
